# baseline (speedup 1.0000x reference)
import functools

import jax
import jax.numpy as jnp
from jax import lax
from jax.experimental import pallas as pl
from jax.experimental.pallas import tpu as pltpu
from jax.experimental.pallas import tpu_sc as plsc

BATCH = 4096
SEQ_LEN = 200
EMBED_DIM = 64
PAD_DIM = 128
LANES = 16

NUM_CORES = 2
NUM_SUBCORES = 16
NUM_WORKERS = NUM_CORES * NUM_SUBCORES
BW = BATCH // NUM_WORKERS
GROUPS = BW // LANES
POS_ROWS = SEQ_LEN * EMBED_DIM
STAGE = POS_ROWS // NUM_SUBCORES
NBUF = 3


def kernel(inputs, pos_table, tok_table):
    idx_t = inputs.T.astype(jnp.int32)
    tok_pad = jnp.pad(tok_table, ((0, 0), (0, PAD_DIM - EMBED_DIM)))
    pos_bc = jnp.broadcast_to(pos_table.reshape(POS_ROWS, 1), (POS_ROWS, PAD_DIM))
    posidx = jnp.arange(POS_ROWS, dtype=jnp.int32)

    mesh = plsc.VectorSubcoreMesh(core_axis_name="c", subcore_axis_name="s")

    @functools.partial(
        pl.kernel,
        out_type=jax.ShapeDtypeStruct((SEQ_LEN, EMBED_DIM, BATCH), jnp.float32),
        mesh=mesh,
        scratch_types=[
            pltpu.VMEM((SEQ_LEN, BW), jnp.int32),
            pltpu.VMEM((POS_ROWS,), jnp.int32),
            [pltpu.VMEM((BW, PAD_DIM), jnp.float32)
             for _ in range(NBUF)],
            [pltpu.VMEM((EMBED_DIM, BW), jnp.float32)
             for _ in range(NBUF)],
            [pltpu.SemaphoreType.DMA for _ in range(NBUF)],
            [pltpu.SemaphoreType.DMA for _ in range(NBUF)],
            [pltpu.SemaphoreType.DMA for _ in range(NBUF)],
        ],
        compiler_params=pltpu.CompilerParams(use_tc_tiling_on_sc=True,
                                             needs_layout_passes=False),
    )
    def embed(idx_hbm, posidx_hbm, posbc_hbm, tok_hbm, out_hbm,
              idx_v, posidx_v, gbuf, tbuf, gsem, psem, ssem):
        sid = lax.axis_index("s")
        wid = sid * NUM_CORES + lax.axis_index("c")
        b0 = wid * BW

        pltpu.sync_copy(idx_hbm.at[pl.ds(0, SEQ_LEN), pl.ds(b0, BW)], idx_v)
        pltpu.sync_copy(posidx_hbm, posidx_v)

        def start_g(l, r):
            pltpu.async_copy(tok_hbm.at[idx_v.at[l]], gbuf[r], gsem[r])

        def wait_g(r):
            pltpu.make_async_copy(tok_hbm.at[idx_v.at[0]], gbuf[r],
                                  gsem[r]).wait()

        def start_p(l, r):
            pltpu.async_copy(
                posbc_hbm.at[posidx_v.at[pl.ds(l * EMBED_DIM, EMBED_DIM)]],
                tbuf[r], psem[r], add=True)

        def wait_p(r):
            pltpu.make_async_copy(
                posbc_hbm.at[posidx_v.at[pl.ds(0, EMBED_DIM)]],
                tbuf[r], psem[r]).wait()

        def start_s(l, r):
            pltpu.async_copy(tbuf[r],
                             out_hbm.at[l, pl.ds(0, EMBED_DIM), pl.ds(b0, BW)],
                             ssem[r])

        def wait_s(r):
            pltpu.make_async_copy(tbuf[r],
                                  out_hbm.at[0, pl.ds(0, EMBED_DIM),
                                             pl.ds(b0, BW)],
                                  ssem[r]).wait()

        iota16 = lax.broadcasted_iota(jnp.int32, (LANES,), 0)
        b_idx = [iota16 + (g * LANES) for g in range(GROUPS)]

        def transpose(r):
            @pl.loop(0, EMBED_DIM)
            def _(d):
                d_vec = jnp.zeros((LANES,), jnp.int32) + d
                for g in range(GROUPS):
                    v = plsc.load_gather(gbuf[r], [b_idx[g], d_vec])
                    tbuf[r][d, pl.ds(g * LANES, LANES)] = v

        for l in range(NBUF):
            start_g(l, l)

        STEADY = (SEQ_LEN // NBUF) * NBUF

        def body(c, k, when):
            pk = (k + NBUF - 1) % NBUF
            wait_g(k)
            when(c >= NBUF, lambda: wait_s(k))
            transpose(k)
            start_p(c, k)
            when(c + NBUF < SEQ_LEN, lambda: start_g(c + NBUF, k))

            def finish_prev():
                wait_p(pk)
                start_s(c - 1, pk)

            when(c >= 1, finish_prev)

        def traced_when(cond, fn):
            pl.when(cond)(fn)

        def static_when(cond, fn):
            if cond:
                fn()

        @pl.loop(0, STEADY, step=NBUF)
        def _(cc):
            for k in range(NBUF):
                body(cc + k, k, traced_when)

        for c in range(STEADY, SEQ_LEN):
            body(c, c % NBUF, static_when)

        last = (SEQ_LEN - 1) % NBUF
        wait_p(last)
        start_s(SEQ_LEN - 1, last)
        for r in range(NBUF):
            wait_s(r)

    out_phys = embed(idx_t, posidx, pos_bc, tok_pad)
    return jnp.transpose(out_phys, (2, 0, 1))


# speedup vs baseline: 1.2818x; 1.2818x over previous
import functools

import jax
import jax.numpy as jnp
from jax import lax
from jax.experimental import pallas as pl
from jax.experimental.pallas import tpu as pltpu
from jax.experimental.pallas import tpu_sc as plsc

BATCH = 4096
SEQ_LEN = 200
EMBED_DIM = 64
PAD_DIM = 128
LANES = 16

NUM_CORES = 2
NUM_SUBCORES = 16
NUM_WORKERS = NUM_CORES * NUM_SUBCORES
BW = BATCH // NUM_WORKERS
GROUPS = BW // LANES
POS_ROWS = SEQ_LEN * EMBED_DIM
STAGE = POS_ROWS // NUM_SUBCORES
NBUF = 3


def kernel(inputs, pos_table, tok_table):
    idx_t = inputs.T.astype(jnp.int32)
    tok_pad = jnp.pad(tok_table, ((0, 0), (0, PAD_DIM - EMBED_DIM)))
    pos_bc = jnp.broadcast_to(pos_table.reshape(POS_ROWS, 1), (POS_ROWS, PAD_DIM))
    posidx = jnp.arange(POS_ROWS, dtype=jnp.int32)

    mesh = plsc.VectorSubcoreMesh(core_axis_name="c", subcore_axis_name="s")

    @functools.partial(
        pl.kernel,
        out_type=jax.ShapeDtypeStruct((SEQ_LEN, EMBED_DIM, BATCH), jnp.float32),
        mesh=mesh,
        scratch_types=[
            pltpu.VMEM((SEQ_LEN, BW), jnp.int32),
            pltpu.VMEM((POS_ROWS,), jnp.int32),
            [pltpu.VMEM((BW, PAD_DIM), jnp.float32)
             for _ in range(NBUF)],
            [pltpu.VMEM((EMBED_DIM, BW), jnp.float32)
             for _ in range(NBUF)],
            [pltpu.SemaphoreType.DMA for _ in range(NBUF)],
            [pltpu.SemaphoreType.DMA for _ in range(NBUF)],
            [pltpu.SemaphoreType.DMA for _ in range(NBUF)],
        ],
        compiler_params=pltpu.CompilerParams(use_tc_tiling_on_sc=True,
                                             needs_layout_passes=False),
    )
    def embed(idx_hbm, posidx_hbm, posbc_hbm, tok_hbm, out_hbm,
              idx_v, posidx_v, gbuf, tbuf, gsem, psem, ssem):
        sid = lax.axis_index("s")
        wid = sid * NUM_CORES + lax.axis_index("c")
        b0 = wid * BW

        pltpu.sync_copy(idx_hbm.at[pl.ds(0, SEQ_LEN), pl.ds(b0, BW)], idx_v)
        pltpu.sync_copy(posidx_hbm, posidx_v)

        def start_g(l, r):
            pltpu.async_copy(tok_hbm.at[idx_v.at[l]], gbuf[r], gsem[r])

        def wait_g(r):
            pltpu.make_async_copy(tok_hbm.at[idx_v.at[0]], gbuf[r],
                                  gsem[r]).wait()

        def start_p(l, r):
            pltpu.async_copy(
                posbc_hbm.at[posidx_v.at[pl.ds(l * EMBED_DIM, EMBED_DIM)]],
                tbuf[r], psem[r], add=True)

        def wait_p(r):
            pltpu.make_async_copy(
                posbc_hbm.at[posidx_v.at[pl.ds(0, EMBED_DIM)]],
                tbuf[r], psem[r]).wait()

        def start_s(l, r):
            pltpu.async_copy(tbuf[r],
                             out_hbm.at[l, pl.ds(0, EMBED_DIM), pl.ds(b0, BW)],
                             ssem[r])

        def wait_s(r):
            pltpu.make_async_copy(tbuf[r],
                                  out_hbm.at[0, pl.ds(0, EMBED_DIM),
                                             pl.ds(b0, BW)],
                                  ssem[r]).wait()

        iota16 = lax.broadcasted_iota(jnp.int32, (LANES,), 0)
        b_idx = [iota16 + (g * LANES) for g in range(GROUPS)]

        def transpose(r):
            @plsc.parallel_loop(0, EMBED_DIM, step=2, unroll=4)
            def _(d):
                vals = []
                for dd in range(2):
                    d_vec = jnp.zeros((LANES,), jnp.int32) + (d + dd)
                    for g in range(GROUPS):
                        vals.append(plsc.load_gather(gbuf[r],
                                                     [b_idx[g], d_vec]))
                for dd in range(2):
                    for g in range(GROUPS):
                        tbuf[r][d + dd, pl.ds(g * LANES, LANES)] = \
                            vals[dd * GROUPS + g]

        for l in range(NBUF):
            start_g(l, l)

        STEADY = (SEQ_LEN // NBUF) * NBUF

        def body(c, k, when):
            pk = (k + NBUF - 1) % NBUF
            wait_g(k)
            when(c >= NBUF, lambda: wait_s(k))
            transpose(k)
            start_p(c, k)
            when(c + NBUF < SEQ_LEN, lambda: start_g(c + NBUF, k))

            def finish_prev():
                wait_p(pk)
                start_s(c - 1, pk)

            when(c >= 1, finish_prev)

        def traced_when(cond, fn):
            pl.when(cond)(fn)

        def static_when(cond, fn):
            if cond:
                fn()

        @pl.loop(0, STEADY, step=NBUF)
        def _(cc):
            for k in range(NBUF):
                body(cc + k, k, traced_when)

        for c in range(STEADY, SEQ_LEN):
            body(c, c % NBUF, static_when)

        last = (SEQ_LEN - 1) % NBUF
        wait_p(last)
        start_s(SEQ_LEN - 1, last)
        for r in range(NBUF):
            wait_s(r)

    out_phys = embed(idx_t, posidx, pos_bc, tok_pad)
    return jnp.transpose(out_phys, (2, 0, 1))


# speedup vs baseline: 1.8126x; 1.4141x over previous
import functools

import jax
import jax.numpy as jnp
from jax import lax
from jax.experimental import pallas as pl
from jax.experimental.pallas import tpu as pltpu
from jax.experimental.pallas import tpu_sc as plsc

BATCH = 4096
SEQ_LEN = 200
EMBED_DIM = 64
TOTAL = BATCH * SEQ_LEN

NUM_CORES = 2
NUM_SUBCORES = 16
NUM_WORKERS = NUM_CORES * NUM_SUBCORES
PER_WORKER = TOTAL // NUM_WORKERS

CHUNK = SEQ_LEN
NUM_CHUNKS = PER_WORKER // CHUNK
WINDOWS = ((0, 120), (120, 80))
LANES = 16
BW = BATCH // NUM_WORKERS
LPAD = 208
NBUF = 4


def kernel(inputs, pos_table, tok_table):
    idx_t = inputs.T.astype(jnp.int32)
    posidx = jnp.arange(CHUNK, dtype=jnp.int32)

    mesh = plsc.VectorSubcoreMesh(core_axis_name="c", subcore_axis_name="s")

    @functools.partial(
        pl.kernel,
        out_type=jax.ShapeDtypeStruct((BATCH, SEQ_LEN, EMBED_DIM), jnp.float32),
        mesh=mesh,
        scratch_types=[
            pltpu.VMEM((SEQ_LEN, BW), jnp.int32),
            pltpu.VMEM((BW, LPAD), jnp.int32),
            pltpu.VMEM((CHUNK,), jnp.int32),
            [pltpu.VMEM((CHUNK, EMBED_DIM), jnp.float32)
             for _ in range(NBUF)],
            pltpu.VMEM_SHARED((SEQ_LEN, EMBED_DIM), jnp.float32),
            [pltpu.SemaphoreType.DMA for _ in range(NBUF)],
            [pltpu.SemaphoreType.DMA for _ in range(NBUF)],
            [pltpu.SemaphoreType.DMA for _ in range(NBUF)],
        ],
        compiler_params=pltpu.CompilerParams(use_tc_tiling_on_sc=False,
                                             needs_layout_passes=False),
    )
    def embed(idx_hbm, posidx_hbm, pos_hbm, tok_hbm, out_hbm,
              idx_v2, idx_vt, posidx_v, rows, pos_sh, gsem, psem, ssem):
        wid = lax.axis_index("s") * NUM_CORES + lax.axis_index("c")
        seq_base = wid * NUM_CHUNKS
        b0 = wid * BW

        @pl.when(lax.axis_index("s") == 0)
        def _():
            pltpu.sync_copy(pos_hbm, rows[0])
            pltpu.sync_copy(rows[0], pos_sh)

        pltpu.sync_copy(posidx_hbm, posidx_v)
        pltpu.sync_copy(idx_hbm.at[pl.ds(0, SEQ_LEN), pl.ds(b0, BW)], idx_v2)

        iota16 = lax.broadcasted_iota(jnp.int32, (LANES,), 0)
        l_idx = [jnp.minimum(iota16 + lg * LANES, SEQ_LEN - 1)
                 for lg in range(LPAD // LANES)]

        @plsc.parallel_loop(0, BW)
        def _(b):
            b_vec = jnp.zeros((LANES,), jnp.int32) + b
            for lg in range(LPAD // LANES):
                v = plsc.load_gather(idx_v2, [l_idx[lg], b_vec])
                idx_vt[b, pl.ds(lg * LANES, LANES)] = v

        plsc.subcore_barrier()

        def start_g(c, b):
            for off, w in WINDOWS:
                pltpu.async_copy(tok_hbm.at[idx_vt.at[c, pl.ds(off, w)]],
                                 rows[b].at[pl.ds(off, w)], gsem[b])

        def wait_g(b):
            for off, w in WINDOWS:
                pltpu.make_async_copy(tok_hbm.at[idx_vt.at[0, pl.ds(off, w)]],
                                      rows[b].at[pl.ds(off, w)],
                                      gsem[b]).wait()

        def start_p(b):
            for off, w in WINDOWS:
                pltpu.async_copy(pos_sh.at[posidx_v.at[pl.ds(off, w)]],
                                 rows[b].at[pl.ds(off, w)], psem[b],
                                 add=True)

        def wait_p(b):
            for off, w in WINDOWS:
                pltpu.make_async_copy(pos_sh.at[posidx_v.at[pl.ds(off, w)]],
                                      rows[b].at[pl.ds(off, w)],
                                      psem[b]).wait()

        def start_s(c, b):
            pltpu.async_copy(rows[b], out_hbm.at[seq_base + c], ssem[b])

        def wait_s(b):
            pltpu.make_async_copy(rows[b], out_hbm.at[seq_base],
                                  ssem[b]).wait()

        for c in range(3):
            start_g(c, c)
        wait_g(0)
        start_p(0)

        @pl.loop(0, NUM_CHUNKS, step=NBUF)
        def _(cc):
            for b in range(NBUF):
                c = cc + b
                b1 = (b + 1) % NBUF

                @pl.when(c + 1 < NUM_CHUNKS)
                def _():
                    wait_g(b1)
                    start_p(b1)

                wait_p(b)
                start_s(c, b)

                b3 = (b + 3) % NBUF

                @pl.when(c + 3 < NUM_CHUNKS)
                def _():
                    @pl.when(c >= 1)
                    def _():
                        wait_s(b3)

                    start_g(c + 3, b3)

        for b in range(NBUF):
            wait_s(b)

    return embed(idx_t, posidx, pos_table, tok_table)


# speedup vs baseline: 1.8301x; 1.0097x over previous
import functools

import jax
import jax.numpy as jnp
from jax import lax
from jax.experimental import pallas as pl
from jax.experimental.pallas import tpu as pltpu
from jax.experimental.pallas import tpu_sc as plsc

BATCH = 4096
SEQ_LEN = 200
EMBED_DIM = 64
TOTAL = BATCH * SEQ_LEN

NUM_CORES = 2
NUM_SUBCORES = 16
NUM_WORKERS = NUM_CORES * NUM_SUBCORES
PER_WORKER = TOTAL // NUM_WORKERS

CHUNK = SEQ_LEN
NUM_CHUNKS = PER_WORKER // CHUNK
IDX_W = 100
IDX_ROWS = CHUNK // IDX_W
IDX_ALL = PER_WORKER // IDX_W
NBUF = 4


def kernel(inputs, pos_table, tok_table):
    idx2d = inputs.reshape(TOTAL // IDX_W, IDX_W).astype(jnp.int32)
    posidx = jnp.arange(CHUNK, dtype=jnp.int32).reshape(IDX_ROWS, IDX_W)

    mesh = plsc.VectorSubcoreMesh(core_axis_name="c", subcore_axis_name="s")

    @functools.partial(
        pl.kernel,
        out_type=jax.ShapeDtypeStruct((BATCH, SEQ_LEN, EMBED_DIM), jnp.float32),
        mesh=mesh,
        scratch_types=[
            pltpu.VMEM((IDX_ALL, IDX_W), jnp.int32),
            pltpu.VMEM((IDX_ROWS, IDX_W), jnp.int32),
            [pltpu.VMEM((CHUNK, EMBED_DIM), jnp.float32)
             for _ in range(NBUF)],
            pltpu.VMEM_SHARED((SEQ_LEN, EMBED_DIM), jnp.float32),
            [pltpu.SemaphoreType.DMA for _ in range(NBUF)],
            [pltpu.SemaphoreType.DMA for _ in range(NBUF)],
            [pltpu.SemaphoreType.DMA for _ in range(NBUF)],
        ],
        compiler_params=pltpu.CompilerParams(use_tc_tiling_on_sc=False),
    )
    def embed(idx_hbm, posidx_hbm, pos_hbm, tok_hbm, out_hbm,
              idx_v, posidx_v, rows, pos_sh, gsem, psem, ssem):
        wid = lax.axis_index("s") * NUM_CORES + lax.axis_index("c")
        seq_base = wid * NUM_CHUNKS
        idx_base = wid * IDX_ALL

        @pl.when(lax.axis_index("s") == 0)
        def _():
            pltpu.sync_copy(pos_hbm, rows[0])
            pltpu.sync_copy(rows[0], pos_sh)

        pltpu.sync_copy(posidx_hbm, posidx_v)
        pltpu.sync_copy(idx_hbm.at[pl.ds(idx_base, IDX_ALL)], idx_v)
        plsc.subcore_barrier()

        def start_g(c, b):
            for j in range(IDX_ROWS):
                pltpu.async_copy(tok_hbm.at[idx_v.at[c * IDX_ROWS + j]],
                                 rows[b].at[pl.ds(j * IDX_W, IDX_W)], gsem[b])

        def wait_g(b):
            for j in range(IDX_ROWS):
                pltpu.make_async_copy(tok_hbm.at[idx_v.at[j]],
                                      rows[b].at[pl.ds(j * IDX_W, IDX_W)],
                                      gsem[b]).wait()

        def start_p(b):
            for j in range(IDX_ROWS):
                pltpu.async_copy(pos_sh.at[posidx_v.at[j]],
                                 rows[b].at[pl.ds(j * IDX_W, IDX_W)], psem[b],
                                 add=True)

        def wait_p(b):
            for j in range(IDX_ROWS):
                pltpu.make_async_copy(pos_sh.at[posidx_v.at[j]],
                                      rows[b].at[pl.ds(j * IDX_W, IDX_W)],
                                      psem[b]).wait()

        def start_s(c, b):
            pltpu.async_copy(rows[b], out_hbm.at[seq_base + c], ssem[b])

        def wait_s(b):
            pltpu.make_async_copy(rows[b], out_hbm.at[seq_base],
                                  ssem[b]).wait()

        for c in range(3):
            start_g(c, c)
        wait_g(0)
        start_p(0)

        @pl.loop(0, NUM_CHUNKS, step=NBUF)
        def _(cc):
            for b in range(NBUF):
                c = cc + b
                b1 = (b + 1) % NBUF

                @pl.when(c + 1 < NUM_CHUNKS)
                def _():
                    wait_g(b1)
                    start_p(b1)

                wait_p(b)
                start_s(c, b)

                b3 = (b + 3) % NBUF

                @pl.when(c + 3 < NUM_CHUNKS)
                def _():
                    @pl.when(c >= 1)
                    def _():
                        wait_s(b3)

                    start_g(c + 3, b3)

        for b in range(NBUF):
            wait_s(b)

    return embed(idx2d, posidx, pos_table, tok_table)
